# Initial kernel scaffold; baseline (speedup 1.0000x reference)
import jax
import jax.numpy as jnp
from jax.experimental import pallas as pl


def kernel(x, edge_index, hyperedge_attr, W_lin, b_lin, W_sheaf, b_sheaf, W1, b1, W2, b2):
    raise NotImplementedError("write your pallas kernel here")


# speedup vs baseline: 45.3388x; 45.3388x over previous
import functools

import jax
import jax.numpy as jnp
from jax import lax
from jax.experimental import pallas as pl
from jax.experimental.pallas import tpu as pltpu
from jax.experimental.pallas import tpu_sc as plsc

N_N = 10000
E_H = 10000
NNZ = 320000
D_F = 128
HID = 64
STK = 6
OUT_C = 128

NW = 32
CH = 128
NNZ_PAD = 323584
PER_W = NNZ_PAD // NW
NCHUNK = PER_W // CH
ZR = 200
TW = 128
PW = 112

_MESH = plsc.VectorSubcoreMesh(core_axis_name="c", subcore_axis_name="s")

_GDN = lax.GatherDimensionNumbers(
    offset_dims=(), collapsed_slice_dims=(0,), start_index_map=(0,))


def _bcast_lane(vec, lane):
    idx = jnp.full((16, 1), lane, jnp.int32)
    return lax.gather(vec, idx, _GDN, (1,),
                      mode=lax.GatherScatterMode.PROMISE_IN_BOUNDS)


@functools.partial(
    pl.kernel,
    out_type=jax.ShapeDtypeStruct((NNZ_PAD * 16,), jnp.float32),
    mesh=_MESH,
    compiler_params=pltpu.CompilerParams(needs_layout_passes=False),
    scratch_types=[
        pltpu.VMEM((N_N * 6,), jnp.float32),
        pltpu.VMEM((E_H * 6,), jnp.float32),
        pltpu.VMEM((CH,), jnp.int32),
        pltpu.VMEM((CH,), jnp.int32),
        pltpu.VMEM((CH * 16,), jnp.float32),
    ],
)
def _sc_h(a_hbm, b_hbm, row_hbm, col_hbm, h_hbm, a_v, b_v, ridx, cidx, hbuf):
    cid = lax.axis_index("c")
    sid = lax.axis_index("s")
    wid = sid * 2 + cid
    lanes = lax.iota(jnp.int32, 16)

    pltpu.sync_copy(a_hbm, a_v)
    pltpu.sync_copy(b_hbm, b_v)

    def zrow(i, _):
        hbuf[pl.ds(i * 16, 16)] = jnp.zeros((16,), jnp.float32)
        return 0

    lax.fori_loop(0, CH, zrow, 0)

    def chunk(k, _):
        start = wid * PER_W + k * CH
        pltpu.sync_copy(row_hbm.at[pl.ds(start, CH)], ridx)
        pltpu.sync_copy(col_hbm.at[pl.ds(start, CH)], cidx)
        for j in range(CH // 16):
            rv = ridx[pl.ds(j * 16, 16)] * 6
            cv = cidx[pl.ds(j * 16, 16)] * 6
            giv = jnp.full((16,), start + j * 16, jnp.int32) + lanes
            validf = jnp.where(giv < NNZ, 1.0, 0.0)
            obase = lanes * 16 + (j * 256)
            for d in range(6):
                av = plsc.load_gather(a_v, (rv + d,))
                bv = plsc.load_gather(b_v, (cv + d,))
                sg = validf / (1.0 + jnp.exp(-(av + bv)))
                plsc.store_scatter(hbuf, (obase + d,), sg)
            plsc.store_scatter(hbuf, (obase + 6,), validf)
        pltpu.sync_copy(hbuf, h_hbm.at[pl.ds(start * 16, CH * 16)])
        return 0

    lax.fori_loop(0, NCHUNK, chunk, 0)


def _make_sc_pass(out_rows, b6):
    nz = out_rows // ZR

    @functools.partial(
        pl.kernel,
        out_type=jax.ShapeDtypeStruct((2, out_rows, PW), jnp.float32),
        mesh=_MESH,
        compiler_params=pltpu.CompilerParams(
            needs_layout_passes=False, use_tc_tiling_on_sc=False),
        scratch_types=[
            pltpu.VMEM((CH,), jnp.int32),
            pltpu.VMEM((CH,), jnp.int32),
            pltpu.VMEM((CH * 16,), jnp.float32),
            pltpu.VMEM((CH, TW), jnp.float32),
            pltpu.VMEM((CH, PW), jnp.float32),
            pltpu.VMEM((ZR, PW), jnp.float32),
            pltpu.VMEM_SHARED((out_rows, PW), jnp.float32),
            pltpu.SemaphoreType.DMA,
        ],
    )
    def k(src_hbm, gidx_hbm, sidx_hbm, h_hbm, out_hbm,
          gidx, sidx, hbuf, srows, pay, zbuf, acc, sem):
        cid = lax.axis_index("c")
        sid = lax.axis_index("s")
        wid = sid * 2 + cid
        lanes = lax.iota(jnp.int32, 16)
        zv = jnp.zeros((16,), jnp.float32)

        def zrow(i, _):
            rowv = jnp.full((16,), i, jnp.int32)
            for dblk in range(PW // 16):
                plsc.store_scatter(zbuf, (rowv, lanes + dblk * 16), zv)
            return 0

        lax.fori_loop(0, ZR, zrow, 0)

        def zpay(i, _):
            plsc.store_scatter(pay, (jnp.full((16,), i, jnp.int32), lanes + 96), zv)
            return 0

        lax.fori_loop(0, CH, zpay, 0)
        for z in range((nz + 15) // 16):
            c = sid + z * 16

            @pl.when(c < nz)
            def _():
                pltpu.sync_copy(zbuf, acc.at[pl.ds(c * ZR, ZR)])

        plsc.subcore_barrier()

        def chunk(kk, _):
            start = wid * PER_W + kk * CH
            pltpu.sync_copy(gidx_hbm.at[pl.ds(start, CH)], gidx)
            pltpu.sync_copy(sidx_hbm.at[pl.ds(start, CH)], sidx)
            pltpu.sync_copy(h_hbm.at[pl.ds(start * 16, CH * 16)], hbuf)
            pltpu.async_copy(src_hbm.at[gidx], srows, sem).wait()

            def inner(i, _):
                hv = hbuf[pl.ds(i * 16, 16)]
                rowv = jnp.full((16,), i, jnp.int32)
                for dblk in range(6):
                    bc = _bcast_lane(hv, dblk)
                    sv = plsc.load_gather(srows, (rowv, lanes + dblk * 16))
                    plsc.store_scatter(pay, (rowv, lanes + dblk * 16), bc * sv)
                if b6 == "ones":
                    plsc.store_scatter(pay, (rowv, lanes + 96), _bcast_lane(hv, 6))
                elif b6 == "hv":
                    plsc.store_scatter(pay, (rowv, lanes + 96), hv)
                elif b6 == "hvsrc":
                    sv6 = plsc.load_gather(srows, (rowv, lanes + 96))
                    plsc.store_scatter(pay, (rowv, lanes + 96), hv * sv6)
                return 0

            lax.fori_loop(0, CH, inner, 0)
            pltpu.sync_copy(pay, acc.at[sidx], add=True)
            return 0

        lax.fori_loop(0, NCHUNK, chunk, 0)
        plsc.subcore_barrier()
        for z in range((nz + 15) // 16):
            c = sid + z * 16

            @pl.when(c < nz)
            def _():
                sl = pl.ds(c * ZR, ZR)
                pltpu.sync_copy(acc.at[sl], out_hbm.at[cid, sl])

    return k


_sc_l1p1 = _make_sc_pass(E_H, None)
_sc_l1p2 = _make_sc_pass(N_N, "ones")
_sc_l2p1 = _make_sc_pass(E_H, "hv")
_sc_l2p2 = _make_sc_pass(N_N, "hvsrc")


RB = 1000


def _pre_body(x_ref, he_ref, wx_ref, cx_ref, wb_ref, cb_ref, y1_ref, bp_ref):
    y1_ref[...] = jnp.dot(x_ref[...], wx_ref[...],
                          preferred_element_type=jnp.float32) + cx_ref[...]
    bp_ref[...] = jnp.dot(he_ref[...], wb_ref[...],
                          preferred_element_type=jnp.float32) + cb_ref[...]


def _comb_body(p_ref, o_ref):
    s = p_ref[0] + p_ref[1]
    o_ref[...] = jnp.concatenate(
        [s, jnp.zeros((s.shape[0], TW - PW), jnp.float32)], axis=1)


def _fin1_body(p_ref, x1_ref, dinv_ref):
    s = p_ref[0] + p_ref[1]
    deg = s[:, 96:97]
    dinv = jnp.where(deg > 0, 1.0 / deg, 0.0)
    x1_ref[...] = jnp.concatenate(
        [jnp.maximum(s[:, :96] * dinv, 0.0),
         jnp.zeros((s.shape[0], 32), jnp.float32)], axis=1)
    dinv_ref[...] = jnp.broadcast_to(dinv, (dinv.shape[0], 16))


def _post_body(p_ref, dinv_ref, w2_ref, b2_ref, o_ref):
    s = (p_ref[0] + p_ref[1]) * dinv_ref[:, :1]
    for dd in range(6):
        o_ref[:, dd * 128:(dd + 1) * 128] = (
            jnp.dot(s[:, dd * 16:(dd + 1) * 16], w2_ref[...],
                    preferred_element_type=jnp.float32)
            + s[:, 96 + dd:97 + dd] * b2_ref[...])


def _comb(p, rows):
    return pl.pallas_call(
        _comb_body,
        grid=(rows // RB,),
        in_specs=[pl.BlockSpec((2, RB, PW), lambda i: (0, i, 0))],
        out_specs=pl.BlockSpec((RB, TW), lambda i: (i, 0)),
        out_shape=jax.ShapeDtypeStruct((rows, TW), jnp.float32),
    )(p)


def kernel(x, edge_index, hyperedge_attr, W_lin, b_lin, W_sheaf, b_sheaf, W1, b1, W2, b2):
    f32 = jnp.float32
    x = x.astype(f32)
    he = hyperedge_attr.astype(f32)
    row = edge_index[0].astype(jnp.int32)
    col = edge_index[1].astype(jnp.int32)
    row_p = jnp.pad(row, (0, NNZ_PAD - NNZ))
    col_p = jnp.pad(col, (0, NNZ_PAD - NNZ))

    WL = W_lin.reshape(D_F, STK, HID)
    bL = b_lin.reshape(STK, HID)
    W_xs = WL.mean(axis=1)
    b_xs = bL.mean(axis=0)
    Wsa, Wsb = W_sheaf[:HID], W_sheaf[HID:]
    Wa = W_xs @ Wsa
    ca = b_xs @ Wsa + b_sheaf
    Wb = W_xs @ Wsb
    cb = b_xs @ Wsb
    W1f = jnp.einsum("kdh,ho->kdo", WL, W1).reshape(D_F, 96)
    b1f = (bL @ W1 + b1).reshape(96)
    Wx = jnp.concatenate([W1f, Wa, jnp.zeros((D_F, 26), f32)], axis=1)
    cx = jnp.concatenate([b1f, ca, jnp.zeros((26,), f32)]).reshape(1, TW)
    Wbp = jnp.pad(Wb, ((0, 0), (0, 10)))
    cbp = jnp.pad(cb, (0, 10)).reshape(1, 16)

    y1x, b_pad = pl.pallas_call(
        _pre_body,
        grid=(N_N // RB,),
        in_specs=[
            pl.BlockSpec((RB, D_F), lambda i: (i, 0)),
            pl.BlockSpec((RB, D_F), lambda i: (i, 0)),
            pl.BlockSpec((D_F, TW), lambda i: (0, 0)),
            pl.BlockSpec((1, TW), lambda i: (0, 0)),
            pl.BlockSpec((D_F, 16), lambda i: (0, 0)),
            pl.BlockSpec((1, 16), lambda i: (0, 0)),
        ],
        out_specs=[
            pl.BlockSpec((RB, TW), lambda i: (i, 0)),
            pl.BlockSpec((RB, 16), lambda i: (i, 0)),
        ],
        out_shape=[
            jax.ShapeDtypeStruct((N_N, TW), f32),
            jax.ShapeDtypeStruct((E_H, 16), f32),
        ],
    )(x, he, Wx, cx, Wbp, cbp)

    a6 = y1x[:, 96:102].reshape(-1)
    b6v = b_pad[:, :6].reshape(-1)

    h_flat = _sc_h(a6, b6v, row_p, col_p)
    p_m1 = _sc_l1p1(y1x, row_p, col_p, h_flat)
    m1 = _comb(p_m1, E_H)
    p_a1 = _sc_l1p2(m1, col_p, row_p, h_flat)

    x1, dinv_pad = pl.pallas_call(
        _fin1_body,
        grid=(N_N // RB,),
        in_specs=[pl.BlockSpec((2, RB, PW), lambda i: (0, i, 0))],
        out_specs=[
            pl.BlockSpec((RB, TW), lambda i: (i, 0)),
            pl.BlockSpec((RB, 16), lambda i: (i, 0)),
        ],
        out_shape=[
            jax.ShapeDtypeStruct((N_N, TW), f32),
            jax.ShapeDtypeStruct((N_N, 16), f32),
        ],
    )(p_a1)

    p_m2 = _sc_l2p1(x1, row_p, col_p, h_flat)
    m2 = _comb(p_m2, E_H)
    p_a2 = _sc_l2p2(m2, col_p, row_p, h_flat)

    out = pl.pallas_call(
        _post_body,
        grid=(N_N // RB,),
        in_specs=[
            pl.BlockSpec((2, RB, PW), lambda i: (0, i, 0)),
            pl.BlockSpec((RB, 16), lambda i: (i, 0)),
            pl.BlockSpec((16, 128), lambda i: (0, 0)),
            pl.BlockSpec((1, 128), lambda i: (0, 0)),
        ],
        out_specs=pl.BlockSpec((RB, STK * OUT_C), lambda i: (i, 0)),
        out_shape=jax.ShapeDtypeStruct((N_N, STK * OUT_C), f32),
    )(p_a2, dinv_pad, W2, b2.reshape(1, OUT_C))

    return out
